# baseline (speedup 1.0000x reference)
import functools

import jax
import jax.numpy as jnp
from jax import lax
from jax.experimental import pallas as pl
from jax.experimental.pallas import tpu as pltpu
from jax.experimental.pallas import tpu_sc as plsc

D = 32
L = 16
TW = 128
DH = D // 2


@functools.lru_cache(maxsize=None)
def _build(B):
    info = plsc.get_sparse_core_info()
    NC, NS = info.num_cores, info.num_subcores
    NW = NC * NS
    bpw = B // NW
    NG = bpw // L

    mesh = plsc.VectorSubcoreMesh(core_axis_name="c", subcore_axis_name="s")

    @functools.partial(
        pl.kernel,
        mesh=mesh,
        out_type=jax.ShapeDtypeStruct((B,), jnp.float32),
        compiler_params=pltpu.CompilerParams(
            needs_layout_passes=False, disable_bounds_checks=True),
        scratch_types=[
            pltpu.VMEM((bpw,), jnp.int32),
            pltpu.VMEM((bpw,), jnp.int32),
            pltpu.VMEM((L, DH, TW), jnp.float32),
            pltpu.VMEM((L, D, TW), jnp.float32),
            pltpu.VMEM((D, L), jnp.float32),
            pltpu.VMEM((D,), jnp.float32),
            pltpu.VMEM((L,), jnp.float32),
            pltpu.VMEM((bpw,), jnp.float32),
            pltpu.SemaphoreType.DMA,
            pltpu.SemaphoreType.DMA,
        ],
    )
    def sc_kernel(uidx_h, iidx_h, utabT_h, itabT_h, w_h, b_h, out_h,
                  uixv, iixv, ublk, iblk, stage, wv, bv, outv, semu, semi):
        wid = lax.axis_index("s") * NC + lax.axis_index("c")
        base = wid * bpw

        pltpu.sync_copy(uidx_h.at[pl.ds(base, bpw)], uixv)
        pltpu.sync_copy(iidx_h.at[pl.ds(base, bpw)], iixv)
        pltpu.sync_copy(w_h, wv)
        pltpu.sync_copy(b_h, bv)

        w_lo = wv[pl.ds(0, L)]
        w_hi = wv[pl.ds(L, L)]
        bvec = bv[...]
        lane = lax.iota(jnp.int32, L)
        zero = jnp.zeros((), jnp.int32)

        def user_slots(g):
            uvec = uixv[pl.ds(g * L, L)]
            cs = uvec & -TW
            uph = uvec & (TW - 1)
            slots = jnp.zeros((L,), jnp.int32)
            isnew = jnp.zeros((L,), jnp.int32)
            slot = zero
            nfetch = zero
            for j in range(L):
                if j == 0:
                    is_new = jnp.bool_(True)
                else:
                    is_new = cs[j] != cs[j - 1]
                slot = jnp.where(is_new, nfetch, slot)
                nfetch = nfetch + jnp.where(is_new, 1, 0)
                isnew = jnp.where(lane == j, jnp.where(is_new, 1, 0), isnew)
                slots = jnp.where(lane == j, slot, slots)
            return cs, uph, slots, isnew, nfetch

        def issue_user(cs, slots, isnew, h, cond):
            for j in range(L):
                off = pl.multiple_of(cs[j], TW)

                @pl.when((isnew[j] == 1) & cond)
                def _(off=off, j=j):
                    pltpu.async_copy(
                        utabT_h.at[pl.ds(h * DH, DH), pl.ds(off, TW)],
                        ublk.at[slots[j]], semu)

        def wait_user(nfetch):
            def wait_one(k, carry):
                pltpu.make_async_copy(
                    utabT_h.at[pl.ds(0, DH), pl.ds(0, TW)],
                    ublk.at[0], semu).wait()
                return carry
            lax.fori_loop(0, nfetch, wait_one, 0)

        true_ = jnp.bool_(True)

        def group(g, carry):
            ucs, uph, slots, isnew, nfetch = carry

            ivec = iixv[pl.ds(g * L, L)]
            ics = ivec & -TW
            for j in range(L):
                off = pl.multiple_of(ics[j], TW)
                pltpu.async_copy(
                    itabT_h.at[:, pl.ds(off, TW)], iblk.at[j], semi)

            for h in range(2):
                if h == 1:
                    issue_user(ucs, slots, isnew, 1, true_)
                wait_user(nfetch)
                for d in range(DH):
                    dv = jnp.full((L,), d, dtype=jnp.int32)
                    w_d = w_lo[d] if h == 0 else w_hi[d]
                    stage[h * DH + d, :] = (
                        plsc.load_gather(ublk, [slots, dv, uph]) * w_d)

            gn = jnp.minimum(g + 1, NG - 1)
            nxt = user_slots(gn)
            issue_user(nxt[0], nxt[2], nxt[3], 0, g + 1 < NG)

            for j in range(L):
                pltpu.make_async_copy(
                    itabT_h.at[:, pl.ds(0, TW)], iblk.at[j], semi).wait()

            iph = ivec & (TW - 1)
            acc = bvec
            for d in range(D):
                dv = jnp.full((L,), d, dtype=jnp.int32)
                acc = acc + stage[d, :] * plsc.load_gather(iblk, [lane, dv, iph])
            outv[pl.ds(g * L, L)] = acc
            return nxt

        first = user_slots(0)
        issue_user(first[0], first[2], first[3], 0, true_)
        lax.fori_loop(0, NG, group, first)

        pltpu.sync_copy(outv, out_h.at[pl.ds(base, bpw)])

    return sc_kernel


def kernel(user_idx, item_idx, user_table, item_table, W, b):
    B = user_idx.shape[0]
    rb = B.bit_length() - 1 if B & (B - 1) == 0 else B.bit_length()
    rows = lax.iota(jnp.int32, B)
    perm = jnp.sort(((user_idx >> 7) << rb) | rows) & (B - 1)
    su = jnp.take(user_idx, perm)
    si = jnp.take(item_idx, perm)
    out_sorted = _build(B)(
        su, si, user_table.T, item_table.T,
        W.reshape(-1), jnp.broadcast_to(b, (L,)))
    inv = jnp.sort((perm << rb) | rows) & (B - 1)
    return jnp.take(out_sorted, inv)


# speedup vs baseline: 1.0308x; 1.0308x over previous
import functools

import jax
import jax.numpy as jnp
from jax import lax
from jax.experimental import pallas as pl
from jax.experimental.pallas import tpu as pltpu
from jax.experimental.pallas import tpu_sc as plsc

D = 32
L = 16
TW = 128
DH = D // 2


@functools.lru_cache(maxsize=None)
def _build(B):
    info = plsc.get_sparse_core_info()
    NC, NS = info.num_cores, info.num_subcores
    NW = NC * NS
    bpw = B // NW
    NG = bpw // L

    mesh = plsc.VectorSubcoreMesh(core_axis_name="c", subcore_axis_name="s")

    @functools.partial(
        pl.kernel,
        mesh=mesh,
        out_type=jax.ShapeDtypeStruct((B,), jnp.float32),
        compiler_params=pltpu.CompilerParams(
            needs_layout_passes=False, disable_bounds_checks=True),
        scratch_types=[
            pltpu.VMEM((bpw,), jnp.int32),
            pltpu.VMEM((bpw,), jnp.int32),
            pltpu.VMEM((L, DH, TW), jnp.float32),
            pltpu.VMEM((L, D, TW), jnp.float32),
            pltpu.VMEM((D, L), jnp.float32),
            pltpu.VMEM((D,), jnp.float32),
            pltpu.VMEM((L,), jnp.float32),
            pltpu.VMEM((bpw,), jnp.float32),
            pltpu.SemaphoreType.DMA,
            pltpu.SemaphoreType.DMA,
        ],
    )
    def sc_kernel(uidx_h, iidx_h, utabT_h, itabT_h, w_h, b_h, out_h,
                  uixv, iixv, ublk, iblk, stage, wv, bv, outv, semu, semi):
        wid = lax.axis_index("s") * NC + lax.axis_index("c")
        base = wid * bpw

        pltpu.sync_copy(uidx_h.at[pl.ds(base, bpw)], uixv)
        pltpu.sync_copy(iidx_h.at[pl.ds(base, bpw)], iixv)
        pltpu.sync_copy(w_h, wv)
        pltpu.sync_copy(b_h, bv)

        w_lo = wv[pl.ds(0, L)]
        w_hi = wv[pl.ds(L, L)]
        bvec = bv[...]
        lane = lax.iota(jnp.int32, L)
        zero = jnp.zeros((), jnp.int32)

        def user_slots(g):
            uvec = uixv[pl.ds(g * L, L)]
            cs = uvec & -TW
            uph = uvec & (TW - 1)
            slots = jnp.zeros((L,), jnp.int32)
            isnew = jnp.zeros((L,), jnp.int32)
            slot = zero
            nfetch = zero
            for j in range(L):
                if j == 0:
                    is_new = jnp.bool_(True)
                else:
                    is_new = cs[j] != cs[j - 1]
                slot = jnp.where(is_new, nfetch, slot)
                nfetch = nfetch + jnp.where(is_new, 1, 0)
                isnew = jnp.where(lane == j, jnp.where(is_new, 1, 0), isnew)
                slots = jnp.where(lane == j, slot, slots)
            return cs, uph, slots, isnew, nfetch

        def issue_user(cs, slots, isnew, h, cond):
            for j in range(L):
                off = pl.multiple_of(cs[j], TW)

                @pl.when((isnew[j] == 1) & cond)
                def _(off=off, j=j):
                    pltpu.async_copy(
                        utabT_h.at[pl.ds(h * DH, DH), pl.ds(off, TW)],
                        ublk.at[slots[j]], semu)

        def wait_user(nfetch):
            def wait_one(k, carry):
                pltpu.make_async_copy(
                    utabT_h.at[pl.ds(0, DH), pl.ds(0, TW)],
                    ublk.at[0], semu).wait()
                return carry
            lax.fori_loop(0, nfetch, wait_one, 0)

        true_ = jnp.bool_(True)

        def group(g, carry):
            ucs, uph, slots, isnew, nfetch = carry

            ivec = iixv[pl.ds(g * L, L)]
            ics = ivec & -TW
            for j in range(L):
                off = pl.multiple_of(ics[j], TW)
                pltpu.async_copy(
                    itabT_h.at[:, pl.ds(off, TW)], iblk.at[j], semi)

            for h in range(2):
                if h == 1:
                    issue_user(ucs, slots, isnew, 1, true_)
                wait_user(nfetch)
                for d in range(DH):
                    dv = jnp.full((L,), d, dtype=jnp.int32)
                    w_d = w_lo[d] if h == 0 else w_hi[d]
                    stage[h * DH + d, :] = (
                        plsc.load_gather(ublk, [slots, dv, uph]) * w_d)

            gn = jnp.minimum(g + 1, NG - 1)
            nxt = user_slots(gn)
            issue_user(nxt[0], nxt[2], nxt[3], 0, g + 1 < NG)

            for j in range(L):
                pltpu.make_async_copy(
                    itabT_h.at[:, pl.ds(0, TW)], iblk.at[j], semi).wait()

            iph = ivec & (TW - 1)
            acc = bvec
            for d in range(D):
                dv = jnp.full((L,), d, dtype=jnp.int32)
                acc = acc + stage[d, :] * plsc.load_gather(iblk, [lane, dv, iph])
            outv[pl.ds(g * L, L)] = acc
            return nxt

        first = user_slots(0)
        issue_user(first[0], first[2], first[3], 0, true_)
        lax.fori_loop(0, NG, group, first)

        pltpu.sync_copy(outv, out_h.at[pl.ds(base, bpw)])

    return sc_kernel


def kernel(user_idx, item_idx, user_table, item_table, W, b):
    B = user_idx.shape[0]
    rows = lax.iota(jnp.int32, B)
    su, perm = lax.sort_key_val(user_idx, rows)
    si = jnp.take(item_idx, perm)
    out_sorted = _build(B)(
        su, si, user_table.T, item_table.T,
        W.reshape(-1), jnp.broadcast_to(b, (L,)))
    _, out = lax.sort_key_val(perm, out_sorted)
    return out


# speedup vs baseline: 1.0634x; 1.0316x over previous
import functools

import jax
import jax.numpy as jnp
from jax import lax
from jax.experimental import pallas as pl
from jax.experimental.pallas import tpu as pltpu
from jax.experimental.pallas import tpu_sc as plsc

D = 32
L = 16
TW = 128
DH = D // 2


@functools.lru_cache(maxsize=None)
def _build(B):
    info = plsc.get_sparse_core_info()
    NC, NS = info.num_cores, info.num_subcores
    NW = NC * NS
    bpw = B // NW
    NG = bpw // L

    mesh = plsc.VectorSubcoreMesh(core_axis_name="c", subcore_axis_name="s")

    @functools.partial(
        pl.kernel,
        mesh=mesh,
        out_type=jax.ShapeDtypeStruct((NC, B), jnp.float32),
        compiler_params=pltpu.CompilerParams(
            needs_layout_passes=False, disable_bounds_checks=True),
        scratch_types=[
            pltpu.VMEM((bpw,), jnp.int32),
            pltpu.VMEM((bpw,), jnp.int32),
            pltpu.VMEM((NG, L), jnp.int32),
            pltpu.VMEM((L, DH, TW), jnp.float32),
            pltpu.VMEM((L, D, TW), jnp.float32),
            pltpu.VMEM((D, L), jnp.float32),
            pltpu.VMEM((D,), jnp.float32),
            pltpu.VMEM((L,), jnp.float32),
            pltpu.VMEM((L,), jnp.float32),
            pltpu.VMEM((B // NS,), jnp.float32),
            pltpu.VMEM_SHARED((B,), jnp.float32),
            pltpu.SemaphoreType.DMA,
            pltpu.SemaphoreType.DMA,
        ],
    )
    def sc_kernel(uidx_h, iidx_h, perm_h, utabT_h, itabT_h, w_h, b_h, out_h,
                  uixv, iixv, rowv, ublk, iblk, stage, wv, bv, accv, winv,
                  slab, semu, semi):
        cid = lax.axis_index("c")
        sid = lax.axis_index("s")
        wid = sid * NC + cid
        base = wid * bpw
        win = B // NS

        pltpu.sync_copy(uidx_h.at[pl.ds(base, bpw)], uixv)
        pltpu.sync_copy(iidx_h.at[pl.ds(base, bpw)], iixv)
        pltpu.sync_copy(perm_h.at[wid], rowv)
        pltpu.sync_copy(w_h, wv)
        pltpu.sync_copy(b_h, bv)

        def zfill(k, carry):
            winv[pl.ds(k * L, L)] = jnp.zeros((L,), jnp.float32)
            return carry
        lax.fori_loop(0, win // L, zfill, 0)
        pltpu.sync_copy(winv, slab.at[pl.ds(sid * win, win)])
        plsc.subcore_barrier()

        w_lo = wv[pl.ds(0, L)]
        w_hi = wv[pl.ds(L, L)]
        bvec = bv[...]
        lane = lax.iota(jnp.int32, L)
        zero = jnp.zeros((), jnp.int32)

        def user_slots(g):
            uvec = uixv[pl.ds(g * L, L)]
            cs = uvec & -TW
            uph = uvec & (TW - 1)
            slots = jnp.zeros((L,), jnp.int32)
            isnew = jnp.zeros((L,), jnp.int32)
            slot = zero
            nfetch = zero
            for j in range(L):
                if j == 0:
                    is_new = jnp.bool_(True)
                else:
                    is_new = cs[j] != cs[j - 1]
                slot = jnp.where(is_new, nfetch, slot)
                nfetch = nfetch + jnp.where(is_new, 1, 0)
                isnew = jnp.where(lane == j, jnp.where(is_new, 1, 0), isnew)
                slots = jnp.where(lane == j, slot, slots)
            return cs, uph, slots, isnew, nfetch

        def issue_user(cs, slots, isnew, h, cond):
            for j in range(L):
                off = pl.multiple_of(cs[j], TW)

                @pl.when((isnew[j] == 1) & cond)
                def _(off=off, j=j):
                    pltpu.async_copy(
                        utabT_h.at[pl.ds(h * DH, DH), pl.ds(off, TW)],
                        ublk.at[slots[j]], semu)

        def wait_user(nfetch):
            def wait_one(k, carry):
                pltpu.make_async_copy(
                    utabT_h.at[pl.ds(0, DH), pl.ds(0, TW)],
                    ublk.at[0], semu).wait()
                return carry
            lax.fori_loop(0, nfetch, wait_one, 0)

        true_ = jnp.bool_(True)

        def group(g, carry):
            ucs, uph, slots, isnew, nfetch = carry

            ivec = iixv[pl.ds(g * L, L)]
            ics = ivec & -TW
            for j in range(L):
                off = pl.multiple_of(ics[j], TW)
                pltpu.async_copy(
                    itabT_h.at[:, pl.ds(off, TW)], iblk.at[j], semi)

            for h in range(2):
                if h == 1:
                    issue_user(ucs, slots, isnew, 1, true_)
                wait_user(nfetch)
                for d in range(DH):
                    dv = jnp.full((L,), d, dtype=jnp.int32)
                    w_d = w_lo[d] if h == 0 else w_hi[d]
                    stage[h * DH + d, :] = (
                        plsc.load_gather(ublk, [slots, dv, uph]) * w_d)

            gn = jnp.minimum(g + 1, NG - 1)
            nxt = user_slots(gn)
            issue_user(nxt[0], nxt[2], nxt[3], 0, g + 1 < NG)

            for j in range(L):
                pltpu.make_async_copy(
                    itabT_h.at[:, pl.ds(0, TW)], iblk.at[j], semi).wait()

            iph = ivec & (TW - 1)
            acc = bvec
            for d in range(D):
                dv = jnp.full((L,), d, dtype=jnp.int32)
                acc = acc + stage[d, :] * plsc.load_gather(iblk, [lane, dv, iph])
            accv[...] = acc
            pltpu.sync_copy(accv, slab.at[rowv.at[g]])
            return nxt

        first = user_slots(0)
        issue_user(first[0], first[2], first[3], 0, true_)
        lax.fori_loop(0, NG, group, first)

        plsc.subcore_barrier()
        pltpu.sync_copy(slab.at[pl.ds(sid * win, win)],
                        out_h.at[cid, pl.ds(sid * win, win)])

    return sc_kernel


def kernel(user_idx, item_idx, user_table, item_table, W, b):
    B = user_idx.shape[0]
    info = plsc.get_sparse_core_info()
    NW = info.num_cores * info.num_subcores
    rows = lax.iota(jnp.int32, B)
    su, perm = lax.sort_key_val(user_idx, rows)
    si = jnp.take(item_idx, perm)
    out2 = _build(B)(
        su, si, perm.reshape(NW, (B // NW) // L, L), user_table.T,
        item_table.T, W.reshape(-1), jnp.broadcast_to(b, (L,)))
    return out2[0] + out2[1]


# speedup vs baseline: 1.0644x; 1.0010x over previous
import functools

import jax
import jax.numpy as jnp
from jax import lax
from jax.experimental import pallas as pl
from jax.experimental.pallas import tpu as pltpu
from jax.experimental.pallas import tpu_sc as plsc

D = 32
L = 16
TW = 128
DH = D // 2


@functools.lru_cache(maxsize=None)
def _build(B):
    info = plsc.get_sparse_core_info()
    NC, NS = info.num_cores, info.num_subcores
    NW = NC * NS
    bpw = B // NW
    NG = bpw // L

    mesh = plsc.VectorSubcoreMesh(core_axis_name="c", subcore_axis_name="s")

    @functools.partial(
        pl.kernel,
        mesh=mesh,
        out_type=jax.ShapeDtypeStruct((NC, B), jnp.float32),
        compiler_params=pltpu.CompilerParams(
            needs_layout_passes=False, disable_bounds_checks=True),
        scratch_types=[
            pltpu.VMEM((bpw,), jnp.int32),
            pltpu.VMEM((bpw,), jnp.int32),
            pltpu.VMEM((NG, L), jnp.int32),
            pltpu.VMEM((L, DH, TW), jnp.float32),
            pltpu.VMEM((L, D, TW), jnp.float32),
            pltpu.VMEM((D, L), jnp.float32),
            pltpu.VMEM((D,), jnp.float32),
            pltpu.VMEM((L,), jnp.float32),
            pltpu.VMEM((L,), jnp.float32),
            pltpu.VMEM((B // NS,), jnp.float32),
            pltpu.VMEM_SHARED((B,), jnp.float32),
            pltpu.SemaphoreType.DMA,
            pltpu.SemaphoreType.DMA,
        ],
    )
    def sc_kernel(uidx_h, iidx_h, perm_h, utabT_h, itabT_h, w_h, b_h, out_h,
                  uixv, iixv, rowv, ublk, iblk, stage, wv, bv, accv, winv,
                  slab, semu, semi):
        cid = lax.axis_index("c")
        sid = lax.axis_index("s")
        wid = sid * NC + cid
        base = wid * bpw
        win = B // NS

        pltpu.sync_copy(uidx_h.at[pl.ds(base, bpw)], uixv)
        pltpu.sync_copy(iidx_h.at[pl.ds(base, bpw)], iixv)
        pltpu.sync_copy(perm_h.at[wid], rowv)
        pltpu.sync_copy(w_h, wv)
        pltpu.sync_copy(b_h, bv)

        def zfill(k, carry):
            winv[pl.ds(k * L, L)] = jnp.zeros((L,), jnp.float32)
            return carry
        lax.fori_loop(0, win // L, zfill, 0)
        pltpu.sync_copy(winv, slab.at[pl.ds(sid * win, win)])
        plsc.subcore_barrier()

        w_lo = wv[pl.ds(0, L)]
        w_hi = wv[pl.ds(L, L)]
        bvec = bv[...]
        lane = lax.iota(jnp.int32, L)
        zero = jnp.zeros((), jnp.int32)

        def user_slots(g):
            uvec = uixv[pl.ds(g * L, L)]
            cs = uvec & -TW
            uph = uvec & (TW - 1)
            slots = jnp.zeros((L,), jnp.int32)
            isnew = jnp.zeros((L,), jnp.int32)
            slot = zero
            nfetch = zero
            for j in range(L):
                if j == 0:
                    is_new = jnp.bool_(True)
                else:
                    is_new = cs[j] != cs[j - 1]
                slot = jnp.where(is_new, nfetch, slot)
                nfetch = nfetch + jnp.where(is_new, 1, 0)
                isnew = jnp.where(lane == j, jnp.where(is_new, 1, 0), isnew)
                slots = jnp.where(lane == j, slot, slots)
            return cs, uph, slots, isnew, nfetch

        def issue_user(cs, slots, isnew, h, cond):
            for j in range(L):
                off = pl.multiple_of(cs[j], TW)

                @pl.when((isnew[j] == 1) & cond)
                def _(off=off, j=j):
                    pltpu.async_copy(
                        utabT_h.at[pl.ds(h * DH, DH), pl.ds(off, TW)],
                        ublk.at[slots[j]], semu)

        def wait_user(nfetch):
            def wait_one(k, carry):
                pltpu.make_async_copy(
                    utabT_h.at[pl.ds(0, DH), pl.ds(0, TW)],
                    ublk.at[0], semu).wait()
                return carry
            lax.fori_loop(0, nfetch, wait_one, 0)

        true_ = jnp.bool_(True)

        def group(g, carry):
            ucs, uph, slots, isnew, nfetch = carry

            ivec = iixv[pl.ds(g * L, L)]
            ics = ivec & -TW
            for j in range(L):
                off = pl.multiple_of(ics[j], TW)
                pltpu.async_copy(
                    itabT_h.at[:, pl.ds(off, TW)], iblk.at[j], semi)

            for h in range(2):
                if h == 1:
                    issue_user(ucs, slots, isnew, 1, true_)
                wait_user(nfetch)
                for d in range(DH):
                    dv = jnp.full((L,), d, dtype=jnp.int32)
                    w_d = w_lo[d] if h == 0 else w_hi[d]
                    stage[h * DH + d, :] = (
                        plsc.load_gather(ublk, [slots, dv, uph]) * w_d)

            gn = jnp.minimum(g + 1, NG - 1)
            nxt = user_slots(gn)
            issue_user(nxt[0], nxt[2], nxt[3], 0, g + 1 < NG)

            for j in range(L):
                pltpu.make_async_copy(
                    itabT_h.at[:, pl.ds(0, TW)], iblk.at[j], semi).wait()

            iph = ivec & (TW - 1)
            acc = bvec
            for d in range(D):
                dv = jnp.full((L,), d, dtype=jnp.int32)
                acc = acc + stage[d, :] * plsc.load_gather(iblk, [lane, dv, iph])
            accv[...] = acc
            pltpu.sync_copy(accv, slab.at[rowv.at[g]])
            return nxt

        first = user_slots(0)
        issue_user(first[0], first[2], first[3], 0, true_)
        lax.fori_loop(0, NG, group, first)

        plsc.subcore_barrier()
        pltpu.sync_copy(slab.at[pl.ds(sid * win, win)],
                        out_h.at[cid, pl.ds(sid * win, win)])

    return sc_kernel


def kernel(user_idx, item_idx, user_table, item_table, W, b):
    B = user_idx.shape[0]
    info = plsc.get_sparse_core_info()
    NW = info.num_cores * info.num_subcores
    rows = lax.iota(jnp.int32, B)
    su, perm = lax.sort_key_val(user_idx, rows)
    si = jnp.take(item_idx, perm)
    out2 = _build(B)(
        su, si, perm.reshape(NW, (B // NW) // L, L), user_table.T,
        item_table.T, W.reshape(-1), jnp.broadcast_to(b, (L,)))
    return out2.sum(axis=0)
